# baseline (speedup 1.0000x reference)
import functools

import jax
import jax.numpy as jnp
from jax import lax
from jax.experimental import pallas as pl
from jax.experimental.pallas import tpu as pltpu
from jax.experimental.pallas import tpu_sc as plsc

B, A, H_DIM, R_DIM = 4096, 200, 128, 64
HUGE = 1e31

_info = plsc.get_sparse_core_info()
_NC, _NS = _info.num_cores, _info.num_subcores
NW = _NC * _NS
BPW = B // NW
ACH = 40
NCH = A // ACH
NG = 13

_SC_PARAMS = pltpu.CompilerParams(
    use_tc_tiling_on_sc=False, needs_layout_passes=False)


def _rtne_bf16(u):
    return (u + 0x7FFF + ((u >> 16) & 1)) >> 16


def _pack_pairs_u32(x):
    u = jax.lax.bitcast_convert_type(x, jnp.uint32)
    r = _rtne_bf16(u)
    word = r[:, 0:32] | (r[:, 32:64] << 16)
    return jax.lax.bitcast_convert_type(word, jnp.float32)


def _unpack_pairs_f32(w):
    u = jax.lax.bitcast_convert_type(w, jnp.uint32)
    lo = jax.lax.bitcast_convert_type(u << 16, jnp.float32)
    hi = jax.lax.bitcast_convert_type(u & jnp.uint32(0xFFFF0000), jnp.float32)
    return lo, hi


def _pack_body(tt_ref, out_ref):
    t = tt_ref[...]
    out_ref[...] = _pack_pairs_u32(t.T)


def _pack_table(table):
    num_r = table.shape[0]
    cb = 12800
    grid = ((num_r + cb - 1) // cb,)
    return pl.pallas_call(
        _pack_body,
        grid=grid,
        in_specs=[pl.BlockSpec((R_DIM, cb), lambda i: (0, i))],
        out_specs=pl.BlockSpec((cb, R_DIM // 2), lambda i: (i, 0)),
        out_shape=jax.ShapeDtypeStruct((num_r, R_DIM // 2), jnp.float32),
    )(table.T)


def _q_gather(table, q):
    mesh = plsc.VectorSubcoreMesh(core_axis_name="c", subcore_axis_name="s")

    @functools.partial(
        pl.kernel,
        mesh=mesh,
        compiler_params=_SC_PARAMS,
        out_type=jax.ShapeDtypeStruct((B, R_DIM // 2), jnp.float32),
        scratch_types=[
            pltpu.VMEM((BPW,), jnp.int32),
            pltpu.VMEM((BPW, R_DIM // 2), jnp.float32),
            pltpu.SemaphoreType.DMA,
        ],
    )
    def qk(table_hbm, q_hbm, out_hbm, idx_v, rows_v, sem):
        wid = lax.axis_index("s") * _NC + lax.axis_index("c")
        base = wid * BPW
        pltpu.sync_copy(q_hbm.at[pl.ds(base, BPW)], idx_v)
        pltpu.async_copy(table_hbm.at[idx_v], rows_v, sem).wait()
        pltpu.sync_copy(rows_v, out_hbm.at[pl.ds(base, BPW)])

    return qk(table, q)


def _sc_scores(table, r_space_t, x2, mask_t):
    mesh = plsc.VectorSubcoreMesh(core_axis_name="c", subcore_axis_name="s")

    @functools.partial(
        pl.kernel,
        mesh=mesh,
        compiler_params=_SC_PARAMS,
        out_type=jax.ShapeDtypeStruct((B, A), jnp.float32),
        scratch_types=[
            pltpu.VMEM((A, BPW), jnp.int32),
            pltpu.VMEM((BPW, A), jnp.int32),
            pltpu.VMEM((BPW, R_DIM // 2), jnp.float32),
            pltpu.VMEM((A, BPW), jnp.float32),
            pltpu.SMEM((BPW,), jnp.int32),
            pltpu.VMEM((BPW, A), jnp.float32),
            pltpu.VMEM((A, R_DIM // 2), jnp.float32),
            pltpu.VMEM((A, R_DIM // 2), jnp.float32),
            pltpu.SemaphoreType.DMA,
            pltpu.SemaphoreType.DMA,
        ],
    )
    def sk(table_hbm, rspt_hbm, x2_hbm, maskt_hbm, out_hbm,
           rsl, idx_s, x2_s, msl, lens_sm, sc_s, rows0, rows1, sem0, sem1):
        wid = lax.axis_index("s") * _NC + lax.axis_index("c")
        base = wid * BPW
        pltpu.sync_copy(rspt_hbm.at[:, pl.ds(base, BPW)], rsl)
        pltpu.sync_copy(x2_hbm.at[pl.ds(base, BPW)], x2_s)
        pltpu.sync_copy(maskt_hbm.at[:, pl.ds(base, BPW)], msl)

        zero16 = jnp.zeros((16,), jnp.float32)
        lane16 = jnp.arange(16, dtype=jnp.int32)

        def len_body(t, c2):
            i0 = t * 16

            def lacc(a, acc):
                return acc + msl[a, pl.ds(i0, 16)]
            acc = lax.fori_loop(0, A, lacc, zero16)
            acci = acc.astype(jnp.int32)
            for k in range(16):
                lens_sm[i0 + k] = acci[k]
            return c2
        lax.fori_loop(0, BPW // 16, len_body, 0)

        def tr_body(i, c2):
            for g in range(NG):
                a0 = min(16 * g, 184)
                vec = plsc.load_gather(
                    rsl, [a0 + lane16, jnp.full((16,), i, jnp.int32)])
                idx_s[i, pl.ds(a0, 16)] = vec
            return c2
        lax.fori_loop(0, BPW, tr_body, 0)

        def zs_body(i, c2):
            for c in range(NG):
                sc_s[i, pl.ds(min(16 * c, 184), 16)] = zero16
            return c2
        lax.fori_loop(0, BPW, zs_body, 0)

        def zr_body(a, c2):
            for v in range(2):
                rows0[a, pl.ds(16 * v, 16)] = zero16
                rows1[a, pl.ds(16 * v, 16)] = zero16
            return c2
        lax.fori_loop(0, A, zr_body, 0)

        def nchunks(ln):
            return (ln + (ACH - 1)) // ACH

        def issue(i1, buf, sem):
            @pl.when(i1 < BPW)
            def _():
                nch = nchunks(lens_sm[i1])
                for j in range(NCH):
                    @pl.when(j < nch)
                    def _():
                        pltpu.async_copy(
                            table_hbm.at[idx_s.at[i1, pl.ds(j * ACH, ACH)]],
                            buf.at[pl.ds(j * ACH, ACH)],
                            sem,
                        )

        def compute(i, buf, sem):
            ln = lens_sm[i]
            nch = nchunks(ln)
            for j in range(NCH):
                @pl.when(j < nch)
                def _():
                    pltpu.make_async_copy(
                        table_hbm.at[idx_s.at[i, pl.ds(j * ACH, ACH)]],
                        buf.at[pl.ds(j * ACH, ACH)],
                        sem,
                    ).wait()
            x2p0 = plsc.bitcast(x2_s[i, pl.ds(0, 16)], jnp.bfloat16)
            x2p1 = plsc.bitcast(x2_s[i, pl.ds(16, 16)], jnp.bfloat16)
            ng = (ln + 15) >> 4

            def a_body(c, carry2):
                a0 = jnp.minimum(c * 16, 184)
                svec = zero16
                for k in range(16):
                    a = a0 + k
                    r0 = plsc.bitcast(buf[a, pl.ds(0, 16)], jnp.bfloat16)
                    r1 = plsc.bitcast(buf[a, pl.ds(16, 16)], jnp.bfloat16)
                    p = r0 * x2p0 + r1 * x2p1
                    u, v = plsc.unpack(p, format=plsc.PackFormat.INTERLEAVED)
                    svec = jnp.where(lane16 == k, jnp.sum(u + v), svec)
                sc_s[i, pl.ds(a0, 16)] = svec
                return carry2

            lax.fori_loop(0, ng, a_body, 0)

        issue(0, rows0, sem0)

        def pair_body(t, carry):
            i = 2 * t
            issue(i + 1, rows1, sem1)
            compute(i, rows0, sem0)
            issue(i + 2, rows0, sem0)
            compute(i + 1, rows1, sem1)
            return carry

        lax.fori_loop(0, BPW // 2, pair_body, 0)
        pltpu.sync_copy(sc_s, out_hbm.at[pl.ds(base, BPW)])

    return sk(table, r_space_t, x2, mask_t)


def _mlp_body(h_ref, q_ref, w1_ref, b1_ref, w2_ref, b2_ref, x2_ref):
    w1h = w1_ref[0:H_DIM, :]
    qlo, qhi = _unpack_pairs_f32(q_ref[...])
    x = jnp.dot(h_ref[...], w1h, preferred_element_type=jnp.float32)
    x = x + jnp.dot(qlo, w1_ref[H_DIM:H_DIM + 32, :],
                    preferred_element_type=jnp.float32)
    x = x + jnp.dot(qhi, w1_ref[H_DIM + 32:H_DIM + R_DIM, :],
                    preferred_element_type=jnp.float32)
    x = jnp.maximum(x + b1_ref[...], 0.0)
    x2 = (jnp.dot(x, w2_ref[...], preferred_element_type=jnp.float32)
          + b2_ref[...])
    x2_ref[...] = _pack_pairs_u32(x2)


def _mlp(H, Q, W1, b1, W2, b2):
    blk = 512
    grid = (B // blk,)
    return pl.pallas_call(
        _mlp_body,
        grid=grid,
        in_specs=[
            pl.BlockSpec((blk, H_DIM), lambda i: (i, 0)),
            pl.BlockSpec((blk, R_DIM // 2), lambda i: (i, 0)),
            pl.BlockSpec((H_DIM + R_DIM, R_DIM), lambda i: (0, 0)),
            pl.BlockSpec((1, R_DIM), lambda i: (0, 0)),
            pl.BlockSpec((R_DIM, R_DIM), lambda i: (0, 0)),
            pl.BlockSpec((1, R_DIM), lambda i: (0, 0)),
        ],
        out_specs=pl.BlockSpec((blk, R_DIM // 2), lambda i: (i, 0)),
        out_shape=jax.ShapeDtypeStruct((B, R_DIM // 2), jnp.float32),
    )(H, Q, W1, b1.reshape(1, R_DIM), W2, b2.reshape(1, R_DIM))


def _smx_body(s_ref, mt_ref, d_ref, e_ref):
    s = s_ref[...] - (1.0 - mt_ref[...].T) * HUGE
    mx = jnp.max(s, axis=1, keepdims=True)
    e = jnp.exp(s - mx)
    z = jnp.sum(e, axis=1, keepdims=True)
    dist = e / z
    d_ref[...] = dist
    e_ref[...] = -jnp.sum(dist * jnp.log(dist + 1e-20), axis=1, keepdims=True)


def _softmax_entropy(scores, mask_t):
    blk = 256
    grid = (B // blk,)
    dist, ent = pl.pallas_call(
        _smx_body,
        grid=grid,
        in_specs=[
            pl.BlockSpec((blk, A), lambda i: (i, 0)),
            pl.BlockSpec((A, blk), lambda i: (0, i)),
        ],
        out_specs=[
            pl.BlockSpec((blk, A), lambda i: (i, 0)),
            pl.BlockSpec((blk, 1), lambda i: (i, 0)),
        ],
        out_shape=[
            jax.ShapeDtypeStruct((B, A), jnp.float32),
            jax.ShapeDtypeStruct((B, 1), jnp.float32),
        ],
    )(scores, mask_t)
    return dist, ent.reshape(B)


def kernel(q, H, r_space, e_space, action_mask, relation_table, W1, b1, W2, b2):
    del e_space
    q = q.astype(jnp.int32)
    r_space = r_space.astype(jnp.int32)
    table_p = _pack_table(relation_table)
    Qp = _q_gather(table_p, q)
    X2p = _mlp(H, Qp, W1, b1, W2, b2)
    r_space_t = r_space.T
    mask_t = action_mask.T
    scores = _sc_scores(table_p, r_space_t, X2p, mask_t)
    return _softmax_entropy(scores, mask_t)


# speedup vs baseline: 1.0362x; 1.0362x over previous
import functools

import jax
import jax.numpy as jnp
from jax import lax
from jax.experimental import pallas as pl
from jax.experimental.pallas import tpu as pltpu
from jax.experimental.pallas import tpu_sc as plsc

B, A, H_DIM, R_DIM = 4096, 200, 128, 64
HUGE = 1e31

_info = plsc.get_sparse_core_info()
_NC, _NS = _info.num_cores, _info.num_subcores
NW = _NC * _NS
BPW = B // NW
ACH = 40
NCH = A // ACH
NG = 13

_SC_PARAMS = pltpu.CompilerParams(
    use_tc_tiling_on_sc=False, needs_layout_passes=False)


def _rtne_bf16(u):
    return (u + 0x7FFF + ((u >> 16) & 1)) >> 16


def _pack_pairs_u32(x):
    u = jax.lax.bitcast_convert_type(x, jnp.uint32)
    r = _rtne_bf16(u)
    word = r[:, 0:32] | (r[:, 32:64] << 16)
    return jax.lax.bitcast_convert_type(word, jnp.float32)


def _unpack_pairs_f32(w):
    u = jax.lax.bitcast_convert_type(w, jnp.uint32)
    lo = jax.lax.bitcast_convert_type(u << 16, jnp.float32)
    hi = jax.lax.bitcast_convert_type(u & jnp.uint32(0xFFFF0000), jnp.float32)
    return lo, hi


def _pack_body(tt_ref, out_ref):
    t = tt_ref[...]
    out_ref[...] = _pack_pairs_u32(t.T)


def _pack_table(table):
    num_r = table.shape[0]
    cb = 12800
    grid = ((num_r + cb - 1) // cb,)
    return pl.pallas_call(
        _pack_body,
        grid=grid,
        in_specs=[pl.BlockSpec((R_DIM, cb), lambda i: (0, i))],
        out_specs=pl.BlockSpec((cb, R_DIM // 2), lambda i: (i, 0)),
        out_shape=jax.ShapeDtypeStruct((num_r, R_DIM // 2), jnp.float32),
    )(table.T)


def _q_gather(table, q):
    mesh = plsc.VectorSubcoreMesh(core_axis_name="c", subcore_axis_name="s")

    @functools.partial(
        pl.kernel,
        mesh=mesh,
        compiler_params=_SC_PARAMS,
        out_type=jax.ShapeDtypeStruct((B, R_DIM // 2), jnp.float32),
        scratch_types=[
            pltpu.VMEM((BPW,), jnp.int32),
            pltpu.VMEM((BPW, R_DIM // 2), jnp.float32),
            pltpu.SemaphoreType.DMA,
        ],
    )
    def qk(table_hbm, q_hbm, out_hbm, idx_v, rows_v, sem):
        wid = lax.axis_index("s") * _NC + lax.axis_index("c")
        base = wid * BPW
        pltpu.sync_copy(q_hbm.at[pl.ds(base, BPW)], idx_v)
        pltpu.async_copy(table_hbm.at[idx_v], rows_v, sem).wait()
        pltpu.sync_copy(rows_v, out_hbm.at[pl.ds(base, BPW)])

    return qk(table, q)


def _sc_scores(table, r_space, x2, mask):
    mesh = plsc.VectorSubcoreMesh(core_axis_name="c", subcore_axis_name="s")

    @functools.partial(
        pl.kernel,
        mesh=mesh,
        compiler_params=_SC_PARAMS,
        out_type=jax.ShapeDtypeStruct((B, A), jnp.float32),
        scratch_types=[
            pltpu.VMEM((BPW, A), jnp.int32),
            pltpu.VMEM((BPW, R_DIM // 2), jnp.float32),
            pltpu.VMEM((BPW, A), jnp.float32),
            pltpu.SMEM((BPW,), jnp.int32),
            pltpu.VMEM((BPW, A), jnp.float32),
            pltpu.VMEM((A, R_DIM // 2), jnp.float32),
            pltpu.VMEM((A, R_DIM // 2), jnp.float32),
            pltpu.SemaphoreType.DMA,
            pltpu.SemaphoreType.DMA,
        ],
    )
    def sk(table_hbm, rsp_hbm, x2_hbm, mask_hbm, out_hbm,
           idx_s, x2_s, mask_s, lens_sm, sc_s, rows0, rows1, sem0, sem1):
        wid = lax.axis_index("s") * _NC + lax.axis_index("c")
        base = wid * BPW
        pltpu.sync_copy(rsp_hbm.at[pl.ds(base, BPW)], idx_s)
        pltpu.sync_copy(x2_hbm.at[pl.ds(base, BPW)], x2_s)
        pltpu.sync_copy(mask_hbm.at[pl.ds(base, BPW)], mask_s)

        zero16 = jnp.zeros((16,), jnp.float32)
        lane16 = jnp.arange(16, dtype=jnp.int32)

        def len_body(i, c2):
            acc = mask_s[i, pl.ds(0, 16)]
            for g in range(1, 12):
                acc = acc + mask_s[i, pl.ds(16 * g, 16)]
            tail = mask_s[i, pl.ds(184, 16)]
            acc = acc + jnp.where(lane16 >= 8, tail, 0.0)
            lens_sm[i] = jnp.sum(acc).astype(jnp.int32)
            return c2
        lax.fori_loop(0, BPW, len_body, 0)

        def zs_body(i, c2):
            for c in range(NG):
                sc_s[i, pl.ds(min(16 * c, 184), 16)] = zero16
            return c2
        lax.fori_loop(0, BPW, zs_body, 0)

        def zr_body(a, c2):
            for v in range(2):
                rows0[a, pl.ds(16 * v, 16)] = zero16
                rows1[a, pl.ds(16 * v, 16)] = zero16
            return c2
        lax.fori_loop(0, A, zr_body, 0)

        def nchunks(ln):
            return (ln + (ACH - 1)) // ACH

        def issue(i1, buf, sem):
            @pl.when(i1 < BPW)
            def _():
                nch = nchunks(lens_sm[i1])
                for j in range(NCH):
                    @pl.when(j < nch)
                    def _():
                        pltpu.async_copy(
                            table_hbm.at[idx_s.at[i1, pl.ds(j * ACH, ACH)]],
                            buf.at[pl.ds(j * ACH, ACH)],
                            sem,
                        )

        def compute(i, buf, sem):
            ln = lens_sm[i]
            nch = nchunks(ln)
            for j in range(NCH):
                @pl.when(j < nch)
                def _():
                    pltpu.make_async_copy(
                        table_hbm.at[idx_s.at[i, pl.ds(j * ACH, ACH)]],
                        buf.at[pl.ds(j * ACH, ACH)],
                        sem,
                    ).wait()
            x2p0 = plsc.bitcast(x2_s[i, pl.ds(0, 16)], jnp.bfloat16)
            x2p1 = plsc.bitcast(x2_s[i, pl.ds(16, 16)], jnp.bfloat16)
            ng = (ln + 15) >> 4

            def a_body(c, carry2):
                a0 = jnp.minimum(c * 16, 184)
                vs = []
                for k in range(16):
                    a = a0 + k
                    r0 = plsc.bitcast(buf[a, pl.ds(0, 16)], jnp.bfloat16)
                    r1 = plsc.bitcast(buf[a, pl.ds(16, 16)], jnp.bfloat16)
                    p = r0 * x2p0 + r1 * x2p1
                    u, v = plsc.unpack(p, format=plsc.PackFormat.INTERLEAVED)
                    vs.append(u + v)
                for r in range(4):
                    bit = 1 << r
                    cond = (lane16 & bit) == 0
                    sh = lane16 ^ bit
                    nxt = []
                    for k2 in range(len(vs) // 2):
                        xx, yy = vs[2 * k2], vs[2 * k2 + 1]
                        pm = jnp.where(cond, xx, yy)
                        qm = jnp.where(cond, yy, xx)
                        nxt.append(
                            pm + qm.at[sh].get(mode="promise_in_bounds"))
                    vs = nxt
                sc_s[i, pl.ds(a0, 16)] = vs[0]
                return carry2

            lax.fori_loop(0, ng, a_body, 0)

        issue(0, rows0, sem0)

        def pair_body(t, carry):
            i = 2 * t
            issue(i + 1, rows1, sem1)
            compute(i, rows0, sem0)
            issue(i + 2, rows0, sem0)
            compute(i + 1, rows1, sem1)
            return carry

        lax.fori_loop(0, BPW // 2, pair_body, 0)
        pltpu.sync_copy(sc_s, out_hbm.at[pl.ds(base, BPW)])

    return sk(table, r_space, x2, mask)


def _mlp_body(h_ref, q_ref, w1_ref, b1_ref, w2_ref, b2_ref, x2_ref):
    w1h = w1_ref[0:H_DIM, :]
    qlo, qhi = _unpack_pairs_f32(q_ref[...])
    x = jnp.dot(h_ref[...], w1h, preferred_element_type=jnp.float32)
    x = x + jnp.dot(qlo, w1_ref[H_DIM:H_DIM + 32, :],
                    preferred_element_type=jnp.float32)
    x = x + jnp.dot(qhi, w1_ref[H_DIM + 32:H_DIM + R_DIM, :],
                    preferred_element_type=jnp.float32)
    x = jnp.maximum(x + b1_ref[...], 0.0)
    x2 = (jnp.dot(x, w2_ref[...], preferred_element_type=jnp.float32)
          + b2_ref[...])
    x2_ref[...] = _pack_pairs_u32(x2)


def _mlp(H, Q, W1, b1, W2, b2):
    blk = 512
    grid = (B // blk,)
    return pl.pallas_call(
        _mlp_body,
        grid=grid,
        in_specs=[
            pl.BlockSpec((blk, H_DIM), lambda i: (i, 0)),
            pl.BlockSpec((blk, R_DIM // 2), lambda i: (i, 0)),
            pl.BlockSpec((H_DIM + R_DIM, R_DIM), lambda i: (0, 0)),
            pl.BlockSpec((1, R_DIM), lambda i: (0, 0)),
            pl.BlockSpec((R_DIM, R_DIM), lambda i: (0, 0)),
            pl.BlockSpec((1, R_DIM), lambda i: (0, 0)),
        ],
        out_specs=pl.BlockSpec((blk, R_DIM // 2), lambda i: (i, 0)),
        out_shape=jax.ShapeDtypeStruct((B, R_DIM // 2), jnp.float32),
    )(H, Q, W1, b1.reshape(1, R_DIM), W2, b2.reshape(1, R_DIM))


def _smx_body(s_ref, m_ref, d_ref, e_ref):
    s = s_ref[...] - (1.0 - m_ref[...]) * HUGE
    mx = jnp.max(s, axis=1, keepdims=True)
    e = jnp.exp(s - mx)
    z = jnp.sum(e, axis=1, keepdims=True)
    dist = e / z
    d_ref[...] = dist
    e_ref[...] = -jnp.sum(dist * jnp.log(dist + 1e-20), axis=1, keepdims=True)


def _softmax_entropy(scores, mask):
    blk = 256
    grid = (B // blk,)
    dist, ent = pl.pallas_call(
        _smx_body,
        grid=grid,
        in_specs=[
            pl.BlockSpec((blk, A), lambda i: (i, 0)),
            pl.BlockSpec((blk, A), lambda i: (i, 0)),
        ],
        out_specs=[
            pl.BlockSpec((blk, A), lambda i: (i, 0)),
            pl.BlockSpec((blk, 1), lambda i: (i, 0)),
        ],
        out_shape=[
            jax.ShapeDtypeStruct((B, A), jnp.float32),
            jax.ShapeDtypeStruct((B, 1), jnp.float32),
        ],
    )(scores, mask)
    return dist, ent.reshape(B)


def kernel(q, H, r_space, e_space, action_mask, relation_table, W1, b1, W2, b2):
    del e_space
    q = q.astype(jnp.int32)
    r_space = r_space.astype(jnp.int32)
    table_p = _pack_table(relation_table)
    Qp = _q_gather(table_p, q)
    X2p = _mlp(H, Qp, W1, b1, W2, b2)
    scores = _sc_scores(table_p, r_space, X2p, action_mask)
    return _softmax_entropy(scores, action_mask)


# speedup vs baseline: 1.1972x; 1.1553x over previous
import functools

import jax
import jax.numpy as jnp
from jax import lax
from jax.experimental import pallas as pl
from jax.experimental.pallas import tpu as pltpu
from jax.experimental.pallas import tpu_sc as plsc

B, A, H_DIM, R_DIM = 4096, 200, 128, 64
HUGE = 1e31

_info = plsc.get_sparse_core_info()
_NC, _NS = _info.num_cores, _info.num_subcores
NW = _NC * _NS
BPW = B // NW
ACH = 40
NCH = A // ACH
NG = 13

_SC_PARAMS = pltpu.CompilerParams(
    use_tc_tiling_on_sc=False, needs_layout_passes=False)


def _rtne_bf16(u):
    return (u + 0x7FFF + ((u >> 16) & 1)) >> 16


def _pack_pairs_u32(x):
    u = jax.lax.bitcast_convert_type(x, jnp.uint32)
    r = _rtne_bf16(u)
    word = r[:, 0:32] | (r[:, 32:64] << 16)
    return jax.lax.bitcast_convert_type(word, jnp.float32)


def _unpack_pairs_f32(w):
    u = jax.lax.bitcast_convert_type(w, jnp.uint32)
    lo = jax.lax.bitcast_convert_type(u << 16, jnp.float32)
    hi = jax.lax.bitcast_convert_type(u & jnp.uint32(0xFFFF0000), jnp.float32)
    return lo, hi


NRQ = 25600


def _pack_body(t0_ref, t1_ref, t2_ref, t3_ref, out_ref):
    parts = [_pack_pairs_u32(t_ref[...].T)
             for t_ref in (t0_ref, t1_ref, t2_ref, t3_ref)]
    out_ref[...] = jnp.concatenate(parts, axis=1)


def _pack_table(table):
    cb4 = 3200
    grid = (NRQ // cb4,)
    nblk = NRQ // cb4

    def mk_spec(g):
        return pl.BlockSpec((R_DIM, cb4), lambda i, g=g: (0, g * nblk + i))

    out = pl.pallas_call(
        _pack_body,
        grid=grid,
        in_specs=[mk_spec(0), mk_spec(1), mk_spec(2), mk_spec(3)],
        out_specs=pl.BlockSpec((cb4, 128), lambda i: (i, 0)),
        out_shape=jax.ShapeDtypeStruct((NRQ, 128), jnp.float32),
    )(table.T, table.T, table.T, table.T)
    return out.reshape(4 * NRQ, R_DIM // 2)


def _r2k(vec):
    q3 = ((vec >= NRQ).astype(jnp.int32)
          + (vec >= 2 * NRQ).astype(jnp.int32)
          + (vec >= 3 * NRQ).astype(jnp.int32))
    return 4 * vec - (4 * NRQ - 1) * q3


def _q_gather(table, q):
    mesh = plsc.VectorSubcoreMesh(core_axis_name="c", subcore_axis_name="s")

    @functools.partial(
        pl.kernel,
        mesh=mesh,
        compiler_params=_SC_PARAMS,
        out_type=jax.ShapeDtypeStruct((B, R_DIM // 2), jnp.float32),
        scratch_types=[
            pltpu.VMEM((BPW,), jnp.int32),
            pltpu.VMEM((BPW, R_DIM // 2), jnp.float32),
            pltpu.SemaphoreType.DMA,
        ],
    )
    def qk(table_hbm, q_hbm, out_hbm, idx_v, rows_v, sem):
        wid = lax.axis_index("s") * _NC + lax.axis_index("c")
        base = wid * BPW
        pltpu.sync_copy(q_hbm.at[pl.ds(base, BPW)], idx_v)
        for v in range(BPW // 16):
            w = idx_v[pl.ds(16 * v, 16)]
            idx_v[pl.ds(16 * v, 16)] = _r2k(w)
        pltpu.async_copy(table_hbm.at[idx_v], rows_v, sem).wait()
        pltpu.sync_copy(rows_v, out_hbm.at[pl.ds(base, BPW)])

    return qk(table, q)


def _sc_scores(table, r_space, x2, mask):
    mesh = plsc.VectorSubcoreMesh(core_axis_name="c", subcore_axis_name="s")

    @functools.partial(
        pl.kernel,
        mesh=mesh,
        compiler_params=_SC_PARAMS,
        out_type=jax.ShapeDtypeStruct((B, A), jnp.float32),
        scratch_types=[
            pltpu.VMEM((BPW, A), jnp.int32),
            pltpu.VMEM((BPW, R_DIM // 2), jnp.float32),
            pltpu.VMEM((BPW, A), jnp.float32),
            pltpu.SMEM((BPW,), jnp.int32),
            pltpu.VMEM((BPW, A), jnp.float32),
            pltpu.VMEM((A, R_DIM // 2), jnp.float32),
            pltpu.VMEM((A, R_DIM // 2), jnp.float32),
            pltpu.SemaphoreType.DMA,
            pltpu.SemaphoreType.DMA,
        ],
    )
    def sk(table_hbm, rsp_hbm, x2_hbm, mask_hbm, out_hbm,
           idx_s, x2_s, mask_s, lens_sm, sc_s, rows0, rows1, sem0, sem1):
        wid = lax.axis_index("s") * _NC + lax.axis_index("c")
        base = wid * BPW
        pltpu.sync_copy(rsp_hbm.at[pl.ds(base, BPW)], idx_s)
        pltpu.sync_copy(x2_hbm.at[pl.ds(base, BPW)], x2_s)
        pltpu.sync_copy(mask_hbm.at[pl.ds(base, BPW)], mask_s)

        zero16 = jnp.zeros((16,), jnp.float32)
        lane16 = jnp.arange(16, dtype=jnp.int32)

        def len_body(i, c2):
            acc = mask_s[i, pl.ds(0, 16)]
            for g in range(1, 12):
                acc = acc + mask_s[i, pl.ds(16 * g, 16)]
            tail = mask_s[i, pl.ds(184, 16)]
            acc = acc + jnp.where(lane16 >= 8, tail, 0.0)
            lens_sm[i] = jnp.sum(acc).astype(jnp.int32)
            return c2
        lax.fori_loop(0, BPW, len_body, 0)

        def zs_body(i, c2):
            for c in range(NG):
                sc_s[i, pl.ds(min(16 * c, 184), 16)] = zero16
            return c2
        lax.fori_loop(0, BPW, zs_body, 0)

        def zr_body(a, c2):
            for v in range(2):
                rows0[a, pl.ds(16 * v, 16)] = zero16
                rows1[a, pl.ds(16 * v, 16)] = zero16
            return c2
        lax.fori_loop(0, A, zr_body, 0)

        def nchunks(ln):
            return (ln + (ACH - 1)) // ACH

        def issue(i1, buf, sem):
            @pl.when(i1 < BPW)
            def _():
                for v in range(12):
                    w = idx_s[i1, pl.ds(16 * v, 16)]
                    idx_s[i1, pl.ds(16 * v, 16)] = _r2k(w)
                wt = idx_s[i1, pl.ds(184, 16)]
                idx_s[i1, pl.ds(184, 16)] = jnp.where(
                    lane16 >= 8, _r2k(wt), wt)
                nch = nchunks(lens_sm[i1])
                for j in range(NCH):
                    @pl.when(j < nch)
                    def _():
                        pltpu.async_copy(
                            table_hbm.at[idx_s.at[i1, pl.ds(j * ACH, ACH)]],
                            buf.at[pl.ds(j * ACH, ACH)],
                            sem,
                        )

        def compute(i, buf, sem):
            ln = lens_sm[i]
            nch = nchunks(ln)
            for j in range(NCH):
                @pl.when(j < nch)
                def _():
                    pltpu.make_async_copy(
                        table_hbm.at[idx_s.at[i, pl.ds(j * ACH, ACH)]],
                        buf.at[pl.ds(j * ACH, ACH)],
                        sem,
                    ).wait()
            x2p0 = plsc.bitcast(x2_s[i, pl.ds(0, 16)], jnp.bfloat16)
            x2p1 = plsc.bitcast(x2_s[i, pl.ds(16, 16)], jnp.bfloat16)
            ng = (ln + 15) >> 4

            def a_body(c, carry2):
                a0 = jnp.minimum(c * 16, 184)
                vs = []
                for k in range(16):
                    a = a0 + k
                    r0 = plsc.bitcast(buf[a, pl.ds(0, 16)], jnp.bfloat16)
                    r1 = plsc.bitcast(buf[a, pl.ds(16, 16)], jnp.bfloat16)
                    p = r0 * x2p0 + r1 * x2p1
                    u, v = plsc.unpack(p, format=plsc.PackFormat.INTERLEAVED)
                    vs.append(u + v)
                for r in range(4):
                    bit = 1 << r
                    cond = (lane16 & bit) == 0
                    sh = lane16 ^ bit
                    nxt = []
                    for k2 in range(len(vs) // 2):
                        xx, yy = vs[2 * k2], vs[2 * k2 + 1]
                        pm = jnp.where(cond, xx, yy)
                        qm = jnp.where(cond, yy, xx)
                        nxt.append(
                            pm + qm.at[sh].get(mode="promise_in_bounds"))
                    vs = nxt
                sc_s[i, pl.ds(a0, 16)] = vs[0]
                return carry2

            lax.fori_loop(0, ng, a_body, 0)

        issue(0, rows0, sem0)

        def pair_body(t, carry):
            i = 2 * t
            issue(i + 1, rows1, sem1)
            compute(i, rows0, sem0)
            issue(i + 2, rows0, sem0)
            compute(i + 1, rows1, sem1)
            return carry

        lax.fori_loop(0, BPW // 2, pair_body, 0)
        pltpu.sync_copy(sc_s, out_hbm.at[pl.ds(base, BPW)])

    return sk(table, r_space, x2, mask)


def _mlp_body(h_ref, q_ref, w1_ref, b1_ref, w2_ref, b2_ref, x2_ref):
    w1h = w1_ref[0:H_DIM, :]
    qlo, qhi = _unpack_pairs_f32(q_ref[...])
    x = jnp.dot(h_ref[...], w1h, preferred_element_type=jnp.float32)
    x = x + jnp.dot(qlo, w1_ref[H_DIM:H_DIM + 32, :],
                    preferred_element_type=jnp.float32)
    x = x + jnp.dot(qhi, w1_ref[H_DIM + 32:H_DIM + R_DIM, :],
                    preferred_element_type=jnp.float32)
    x = jnp.maximum(x + b1_ref[...], 0.0)
    x2 = (jnp.dot(x, w2_ref[...], preferred_element_type=jnp.float32)
          + b2_ref[...])
    x2_ref[...] = _pack_pairs_u32(x2)


def _mlp(H, Q, W1, b1, W2, b2):
    blk = 512
    grid = (B // blk,)
    return pl.pallas_call(
        _mlp_body,
        grid=grid,
        in_specs=[
            pl.BlockSpec((blk, H_DIM), lambda i: (i, 0)),
            pl.BlockSpec((blk, R_DIM // 2), lambda i: (i, 0)),
            pl.BlockSpec((H_DIM + R_DIM, R_DIM), lambda i: (0, 0)),
            pl.BlockSpec((1, R_DIM), lambda i: (0, 0)),
            pl.BlockSpec((R_DIM, R_DIM), lambda i: (0, 0)),
            pl.BlockSpec((1, R_DIM), lambda i: (0, 0)),
        ],
        out_specs=pl.BlockSpec((blk, R_DIM // 2), lambda i: (i, 0)),
        out_shape=jax.ShapeDtypeStruct((B, R_DIM // 2), jnp.float32),
    )(H, Q, W1, b1.reshape(1, R_DIM), W2, b2.reshape(1, R_DIM))


def _smx_body(s_ref, m_ref, d_ref, e_ref):
    s = s_ref[...] - (1.0 - m_ref[...]) * HUGE
    mx = jnp.max(s, axis=1, keepdims=True)
    e = jnp.exp(s - mx)
    z = jnp.sum(e, axis=1, keepdims=True)
    dist = e / z
    d_ref[...] = dist
    e_ref[...] = -jnp.sum(dist * jnp.log(dist + 1e-20), axis=1, keepdims=True)


def _softmax_entropy(scores, mask):
    blk = 256
    grid = (B // blk,)
    dist, ent = pl.pallas_call(
        _smx_body,
        grid=grid,
        in_specs=[
            pl.BlockSpec((blk, A), lambda i: (i, 0)),
            pl.BlockSpec((blk, A), lambda i: (i, 0)),
        ],
        out_specs=[
            pl.BlockSpec((blk, A), lambda i: (i, 0)),
            pl.BlockSpec((blk, 1), lambda i: (i, 0)),
        ],
        out_shape=[
            jax.ShapeDtypeStruct((B, A), jnp.float32),
            jax.ShapeDtypeStruct((B, 1), jnp.float32),
        ],
    )(scores, mask)
    return dist, ent.reshape(B)


def kernel(q, H, r_space, e_space, action_mask, relation_table, W1, b1, W2, b2):
    del e_space
    q = q.astype(jnp.int32)
    r_space = r_space.astype(jnp.int32)
    table_p = _pack_table(relation_table)
    Qp = _q_gather(table_p, q)
    X2p = _mlp(H, Qp, W1, b1, W2, b2)
    scores = _sc_scores(table_p, r_space, X2p, action_mask)
    return _softmax_entropy(scores, action_mask)


# speedup vs baseline: 1.3586x; 1.1348x over previous
import functools

import jax
import jax.numpy as jnp
from jax import lax
from jax.experimental import pallas as pl
from jax.experimental.pallas import tpu as pltpu
from jax.experimental.pallas import tpu_sc as plsc

B, A, H_DIM, R_DIM = 4096, 200, 128, 64
HUGE = 1e31

_info = plsc.get_sparse_core_info()
_NC, _NS = _info.num_cores, _info.num_subcores
NW = _NC * _NS
BPW = B // NW
ACH = 40
NCH = A // ACH
NG = 13

_SC_PARAMS = pltpu.CompilerParams(
    use_tc_tiling_on_sc=False, needs_layout_passes=False)


def _rtne_bf16(u):
    return (u + 0x7FFF + ((u >> 16) & 1)) >> 16


def _pack_pairs_u32(x):
    u = jax.lax.bitcast_convert_type(x, jnp.uint32)
    r = _rtne_bf16(u)
    word = r[:, 0:32] | (r[:, 32:64] << 16)
    return jax.lax.bitcast_convert_type(word, jnp.float32)


def _unpack_pairs_f32(w):
    u = jax.lax.bitcast_convert_type(w, jnp.uint32)
    lo = jax.lax.bitcast_convert_type(u << 16, jnp.float32)
    hi = jax.lax.bitcast_convert_type(u & jnp.uint32(0xFFFF0000), jnp.float32)
    return lo, hi


NRQ = 25600


def _pack_body(t0_ref, t1_ref, t2_ref, t3_ref, out_ref):
    parts = [_pack_pairs_u32(t_ref[...].T)
             for t_ref in (t0_ref, t1_ref, t2_ref, t3_ref)]
    out_ref[...] = jnp.concatenate(parts, axis=1)


def _pack_table(table):
    cb4 = 3200
    grid = (NRQ // cb4,)
    nblk = NRQ // cb4

    def mk_spec(g):
        return pl.BlockSpec((R_DIM, cb4), lambda i, g=g: (0, g * nblk + i))

    out = pl.pallas_call(
        _pack_body,
        grid=grid,
        in_specs=[mk_spec(0), mk_spec(1), mk_spec(2), mk_spec(3)],
        out_specs=pl.BlockSpec((cb4, 128), lambda i: (i, 0)),
        out_shape=jax.ShapeDtypeStruct((NRQ, 128), jnp.float32),
    )(table.T, table.T, table.T, table.T)
    return out.reshape(4 * NRQ, R_DIM // 2)


def _r2k(vec):
    q3 = ((vec >= NRQ).astype(jnp.int32)
          + (vec >= 2 * NRQ).astype(jnp.int32)
          + (vec >= 3 * NRQ).astype(jnp.int32))
    return 4 * vec - (4 * NRQ - 1) * q3


def _q_gather(table, q):
    mesh = plsc.VectorSubcoreMesh(core_axis_name="c", subcore_axis_name="s")

    @functools.partial(
        pl.kernel,
        mesh=mesh,
        compiler_params=_SC_PARAMS,
        out_type=jax.ShapeDtypeStruct((B, R_DIM // 2), jnp.float32),
        scratch_types=[
            pltpu.VMEM((BPW,), jnp.int32),
            pltpu.VMEM((BPW, R_DIM // 2), jnp.float32),
            pltpu.SemaphoreType.DMA,
        ],
    )
    def qk(table_hbm, q_hbm, out_hbm, idx_v, rows_v, sem):
        wid = lax.axis_index("s") * _NC + lax.axis_index("c")
        base = wid * BPW
        pltpu.sync_copy(q_hbm.at[pl.ds(base, BPW)], idx_v)
        for v in range(BPW // 16):
            w = idx_v[pl.ds(16 * v, 16)]
            idx_v[pl.ds(16 * v, 16)] = _r2k(w)
        pltpu.async_copy(table_hbm.at[idx_v], rows_v, sem).wait()
        pltpu.sync_copy(rows_v, out_hbm.at[pl.ds(base, BPW)])

    return qk(table, q)


def _sc_scores(table, r_space, x2, mask):
    mesh = plsc.VectorSubcoreMesh(core_axis_name="c", subcore_axis_name="s")

    @functools.partial(
        pl.kernel,
        mesh=mesh,
        compiler_params=_SC_PARAMS,
        out_type=jax.ShapeDtypeStruct((B, A), jnp.float32),
        scratch_types=[
            pltpu.VMEM((BPW, A), jnp.int32),
            pltpu.VMEM((BPW, R_DIM // 2), jnp.float32),
            pltpu.VMEM((BPW, A), jnp.float32),
            pltpu.SMEM((BPW,), jnp.int32),
            pltpu.VMEM((BPW, A), jnp.float32),
            pltpu.VMEM((A, R_DIM // 2), jnp.float32),
            pltpu.VMEM((A, R_DIM // 2), jnp.float32),
            pltpu.VMEM((A, R_DIM // 2), jnp.float32),
            pltpu.VMEM((A, R_DIM // 2), jnp.float32),
            pltpu.SemaphoreType.DMA,
            pltpu.SemaphoreType.DMA,
            pltpu.SemaphoreType.DMA,
            pltpu.SemaphoreType.DMA,
        ],
    )
    def sk(table_hbm, rsp_hbm, x2_hbm, mask_hbm, out_hbm,
           idx_s, x2_s, mask_s, lens_sm, sc_s, rows0, rows1, rows2, rows3,
           sem0, sem1, sem2, sem3):
        wid = lax.axis_index("s") * _NC + lax.axis_index("c")
        base = wid * BPW
        pltpu.sync_copy(rsp_hbm.at[pl.ds(base, BPW)], idx_s)
        pltpu.sync_copy(x2_hbm.at[pl.ds(base, BPW)], x2_s)
        pltpu.sync_copy(mask_hbm.at[pl.ds(base, BPW)], mask_s)

        zero16 = jnp.zeros((16,), jnp.float32)
        lane16 = jnp.arange(16, dtype=jnp.int32)

        def len_body(i, c2):
            acc = mask_s[i, pl.ds(0, 16)]
            for g in range(1, 12):
                acc = acc + mask_s[i, pl.ds(16 * g, 16)]
            tail = mask_s[i, pl.ds(184, 16)]
            acc = acc + jnp.where(lane16 >= 8, tail, 0.0)
            lens_sm[i] = jnp.sum(acc).astype(jnp.int32)
            return c2
        lax.fori_loop(0, BPW, len_body, 0)

        def zs_body(i, c2):
            for c in range(NG):
                sc_s[i, pl.ds(min(16 * c, 184), 16)] = zero16
            return c2
        lax.fori_loop(0, BPW, zs_body, 0)

        def zr_body(a, c2):
            for v in range(2):
                rows0[a, pl.ds(16 * v, 16)] = zero16
                rows1[a, pl.ds(16 * v, 16)] = zero16
                rows2[a, pl.ds(16 * v, 16)] = zero16
                rows3[a, pl.ds(16 * v, 16)] = zero16
            return c2
        lax.fori_loop(0, A, zr_body, 0)

        def nchunks(ln):
            return (ln + (ACH - 1)) // ACH

        def issue(i1, buf, sem):
            @pl.when(i1 < BPW)
            def _():
                for v in range(12):
                    w = idx_s[i1, pl.ds(16 * v, 16)]
                    idx_s[i1, pl.ds(16 * v, 16)] = _r2k(w)
                wt = idx_s[i1, pl.ds(184, 16)]
                idx_s[i1, pl.ds(184, 16)] = jnp.where(
                    lane16 >= 8, _r2k(wt), wt)
                nch = nchunks(lens_sm[i1])
                for j in range(NCH):
                    @pl.when(j < nch)
                    def _():
                        pltpu.async_copy(
                            table_hbm.at[idx_s.at[i1, pl.ds(j * ACH, ACH)]],
                            buf.at[pl.ds(j * ACH, ACH)],
                            sem,
                        )

        def compute(i, buf, sem):
            ln = lens_sm[i]
            nch = nchunks(ln)
            for j in range(NCH):
                @pl.when(j < nch)
                def _():
                    pltpu.make_async_copy(
                        table_hbm.at[idx_s.at[i, pl.ds(j * ACH, ACH)]],
                        buf.at[pl.ds(j * ACH, ACH)],
                        sem,
                    ).wait()
            x2p0 = plsc.bitcast(x2_s[i, pl.ds(0, 16)], jnp.bfloat16)
            x2p1 = plsc.bitcast(x2_s[i, pl.ds(16, 16)], jnp.bfloat16)
            ng = (ln + 15) >> 4

            def a_body(c, carry2):
                a0 = jnp.minimum(c * 16, 184)
                vs = []
                for k in range(16):
                    a = a0 + k
                    r0 = plsc.bitcast(buf[a, pl.ds(0, 16)], jnp.bfloat16)
                    r1 = plsc.bitcast(buf[a, pl.ds(16, 16)], jnp.bfloat16)
                    p = r0 * x2p0 + r1 * x2p1
                    u, v = plsc.unpack(p, format=plsc.PackFormat.INTERLEAVED)
                    vs.append(u + v)
                for r in range(4):
                    bit = 1 << r
                    cond = (lane16 & bit) == 0
                    sh = lane16 ^ bit
                    nxt = []
                    for k2 in range(len(vs) // 2):
                        xx, yy = vs[2 * k2], vs[2 * k2 + 1]
                        pm = jnp.where(cond, xx, yy)
                        qm = jnp.where(cond, yy, xx)
                        nxt.append(
                            pm + qm.at[sh].get(mode="promise_in_bounds"))
                    vs = nxt
                sc_s[i, pl.ds(a0, 16)] = vs[0]
                return carry2

            lax.fori_loop(0, ng, a_body, 0)

        issue(0, rows0, sem0)
        issue(1, rows1, sem1)
        issue(2, rows2, sem2)

        def quad_body(t, carry):
            i = 4 * t
            issue(i + 3, rows3, sem3)
            compute(i, rows0, sem0)
            issue(i + 4, rows0, sem0)
            compute(i + 1, rows1, sem1)
            issue(i + 5, rows1, sem1)
            compute(i + 2, rows2, sem2)
            issue(i + 6, rows2, sem2)
            compute(i + 3, rows3, sem3)
            return carry

        lax.fori_loop(0, BPW // 4, quad_body, 0)
        pltpu.sync_copy(sc_s, out_hbm.at[pl.ds(base, BPW)])

    return sk(table, r_space, x2, mask)


def _mlp_body(h_ref, q_ref, w1_ref, b1_ref, w2_ref, b2_ref, x2_ref):
    w1h = w1_ref[0:H_DIM, :]
    qlo, qhi = _unpack_pairs_f32(q_ref[...])
    x = jnp.dot(h_ref[...], w1h, preferred_element_type=jnp.float32)
    x = x + jnp.dot(qlo, w1_ref[H_DIM:H_DIM + 32, :],
                    preferred_element_type=jnp.float32)
    x = x + jnp.dot(qhi, w1_ref[H_DIM + 32:H_DIM + R_DIM, :],
                    preferred_element_type=jnp.float32)
    x = jnp.maximum(x + b1_ref[...], 0.0)
    x2 = (jnp.dot(x, w2_ref[...], preferred_element_type=jnp.float32)
          + b2_ref[...])
    x2_ref[...] = _pack_pairs_u32(x2)


def _mlp(H, Q, W1, b1, W2, b2):
    blk = 512
    grid = (B // blk,)
    return pl.pallas_call(
        _mlp_body,
        grid=grid,
        in_specs=[
            pl.BlockSpec((blk, H_DIM), lambda i: (i, 0)),
            pl.BlockSpec((blk, R_DIM // 2), lambda i: (i, 0)),
            pl.BlockSpec((H_DIM + R_DIM, R_DIM), lambda i: (0, 0)),
            pl.BlockSpec((1, R_DIM), lambda i: (0, 0)),
            pl.BlockSpec((R_DIM, R_DIM), lambda i: (0, 0)),
            pl.BlockSpec((1, R_DIM), lambda i: (0, 0)),
        ],
        out_specs=pl.BlockSpec((blk, R_DIM // 2), lambda i: (i, 0)),
        out_shape=jax.ShapeDtypeStruct((B, R_DIM // 2), jnp.float32),
    )(H, Q, W1, b1.reshape(1, R_DIM), W2, b2.reshape(1, R_DIM))


def _smx_body(s_ref, m_ref, d_ref, e_ref):
    s = s_ref[...] - (1.0 - m_ref[...]) * HUGE
    mx = jnp.max(s, axis=1, keepdims=True)
    e = jnp.exp(s - mx)
    z = jnp.sum(e, axis=1, keepdims=True)
    dist = e / z
    d_ref[...] = dist
    e_ref[...] = -jnp.sum(dist * jnp.log(dist + 1e-20), axis=1, keepdims=True)


def _softmax_entropy(scores, mask):
    blk = 256
    grid = (B // blk,)
    dist, ent = pl.pallas_call(
        _smx_body,
        grid=grid,
        in_specs=[
            pl.BlockSpec((blk, A), lambda i: (i, 0)),
            pl.BlockSpec((blk, A), lambda i: (i, 0)),
        ],
        out_specs=[
            pl.BlockSpec((blk, A), lambda i: (i, 0)),
            pl.BlockSpec((blk, 1), lambda i: (i, 0)),
        ],
        out_shape=[
            jax.ShapeDtypeStruct((B, A), jnp.float32),
            jax.ShapeDtypeStruct((B, 1), jnp.float32),
        ],
    )(scores, mask)
    return dist, ent.reshape(B)


def kernel(q, H, r_space, e_space, action_mask, relation_table, W1, b1, W2, b2):
    del e_space
    q = q.astype(jnp.int32)
    r_space = r_space.astype(jnp.int32)
    table_p = _pack_table(relation_table)
    Qp = _q_gather(table_p, q)
    X2p = _mlp(H, Qp, W1, b1, W2, b2)
    scores = _sc_scores(table_p, r_space, X2p, action_mask)
    return _softmax_entropy(scores, action_mask)
